# Initial kernel scaffold; baseline (speedup 1.0000x reference)
import jax
import jax.numpy as jnp
from jax.experimental import pallas as pl


def kernel(x, gt, aug_rand_info):
    raise NotImplementedError("write your pallas kernel here")


# speedup vs baseline: 9.9462x; 9.9462x over previous
import jax
import jax.numpy as jnp
from jax.experimental import pallas as pl

_CBLK = 128
_S = 4096


def _body(x_ref, gt_ref, w_ref, o_ref):
    K = w_ref.shape[1]
    HW = x_ref.shape[2]
    f32 = jnp.float32
    nch = HW // _S
    hp = jax.lax.Precision.HIGHEST

    def onehot(i):
        gt_s = gt_ref[0, :, pl.ds(i * _S, _S)]
        cls = jax.lax.broadcasted_iota(jnp.int32, (K, _S), 0)
        return (cls == gt_s).astype(f32)

    fsum = jnp.zeros((K, _CBLK), f32)
    fsq = jnp.zeros((K, _CBLK), f32)
    cnt = jnp.zeros((K, 1), f32)
    for i in range(nch):
        oh = onehot(i)
        xc = x_ref[0, :, pl.ds(i * _S, _S)]
        fsum = fsum + jax.lax.dot_general(
            oh, xc, (((1,), (1,)), ((), ())), preferred_element_type=f32)
        fsq = fsq + jax.lax.dot_general(
            oh, xc * xc, (((1,), (1,)), ((), ())), preferred_element_type=f32)
        cnt = cnt + jnp.sum(oh, axis=1, keepdims=True)

    rc = 1.0 / jnp.where(cnt > 0, cnt, 1.0)
    mean = fsum * rc
    var = jnp.maximum(fsq * rc - mean * mean, 0.0)
    std = jnp.sqrt(var) + 1e-7
    wm = w_ref[0]
    sm = jax.lax.dot_general(
        wm, mean, (((1,), (0,)), ((), ())), precision=hp,
        preferred_element_type=f32)
    ss = jax.lax.dot_general(
        wm, std, (((1,), (0,)), ((), ())), precision=hp,
        preferred_element_type=f32)
    rss = ss / std

    def gather(tbl, oh):
        return jax.lax.dot_general(
            tbl, oh, (((0,), (0,)), ((), ())), precision=hp,
            preferred_element_type=f32)

    for i in range(nch):
        oh = onehot(i)
        xc = x_ref[0, :, pl.ds(i * _S, _S)]
        mg = gather(mean, oh)
        rg = gather(rss, oh)
        sg = gather(sm, oh)
        o_ref[0, :, pl.ds(i * _S, _S)] = (xc - mg) * rg + sg


def kernel(x, gt, aug_rand_info):
    B, C, H, W = x.shape
    K = aug_rand_info.shape[1]
    HW = H * W
    xf = x.reshape(B, C, HW)
    gtf = gt.reshape(B, 1, HW).astype(jnp.int32)
    w = aug_rand_info.reshape(B, K, K)
    nc = C // _CBLK
    xs = pl.pallas_call(
        _body,
        grid=(B, nc),
        in_specs=[
            pl.BlockSpec((1, _CBLK, HW), lambda b, c: (b, c, 0)),
            pl.BlockSpec((1, 1, HW), lambda b, c: (b, 0, 0)),
            pl.BlockSpec((1, K, K), lambda b, c: (b, 0, 0)),
        ],
        out_specs=pl.BlockSpec((1, _CBLK, HW), lambda b, c: (b, c, 0)),
        out_shape=jax.ShapeDtypeStruct((B, C, HW), x.dtype),
    )(xf, gtf, w)
    return (x, xs.reshape(B, C, H, W))


# speedup vs baseline: 15.6933x; 1.5778x over previous
import jax
import jax.numpy as jnp
from jax.experimental import pallas as pl

_CBLK = 128
_S = 4096


def _body(x_ref, gt_ref, w_ref, o_ref):
    K = w_ref.shape[1]
    HW = x_ref.shape[2]
    f32 = jnp.float32
    nch = HW // _S
    hp = jax.lax.Precision.HIGHEST

    def onehot(i):
        gt_s = gt_ref[0, :, pl.ds(i * _S, _S)]
        cls = jax.lax.broadcasted_iota(jnp.int32, (K, _S), 0)
        return (cls == gt_s).astype(f32)

    fsum = jnp.zeros((K, _CBLK), f32)
    fsq = jnp.zeros((K, _CBLK), f32)
    cnt = jnp.zeros((K, 1), f32)
    for i in range(nch):
        oh = onehot(i)
        xc = x_ref[0, :, pl.ds(i * _S, _S)]
        fsum = fsum + jax.lax.dot_general(
            oh, xc, (((1,), (1,)), ((), ())), preferred_element_type=f32)
        fsq = fsq + jax.lax.dot_general(
            oh, xc * xc, (((1,), (1,)), ((), ())), preferred_element_type=f32)
        cnt = cnt + jnp.sum(oh, axis=1, keepdims=True)

    rc = 1.0 / jnp.where(cnt > 0, cnt, 1.0)
    mean = fsum * rc
    var = jnp.maximum(fsq * rc - mean * mean, 0.0)
    std = jnp.sqrt(var) + 1e-7
    wm = w_ref[0]
    sm = jax.lax.dot_general(
        wm, mean, (((1,), (0,)), ((), ())), precision=hp,
        preferred_element_type=f32)
    ss = jax.lax.dot_general(
        wm, std, (((1,), (0,)), ((), ())), precision=hp,
        preferred_element_type=f32)
    rss = ss / std

    def gather(tbl, oh):
        return jax.lax.dot_general(
            tbl, oh, (((0,), (0,)), ((), ())),
            preferred_element_type=f32)

    for i in range(nch):
        oh = onehot(i)
        xc = x_ref[0, :, pl.ds(i * _S, _S)]
        mg = gather(mean, oh)
        rg = gather(rss, oh)
        sg = gather(sm, oh)
        o_ref[0, :, pl.ds(i * _S, _S)] = (xc - mg) * rg + sg


def kernel(x, gt, aug_rand_info):
    B, C, H, W = x.shape
    K = aug_rand_info.shape[1]
    HW = H * W
    xf = x.reshape(B, C, HW)
    gtf = gt.reshape(B, 1, HW).astype(jnp.int32)
    w = aug_rand_info.reshape(B, K, K)
    nc = C // _CBLK
    xs = pl.pallas_call(
        _body,
        grid=(B, nc),
        in_specs=[
            pl.BlockSpec((1, _CBLK, HW), lambda b, c: (b, c, 0)),
            pl.BlockSpec((1, 1, HW), lambda b, c: (b, 0, 0)),
            pl.BlockSpec((1, K, K), lambda b, c: (b, 0, 0)),
        ],
        out_specs=pl.BlockSpec((1, _CBLK, HW), lambda b, c: (b, c, 0)),
        out_shape=jax.ShapeDtypeStruct((B, C, HW), x.dtype),
    )(xf, gtf, w)
    return (x, xs.reshape(B, C, H, W))
